# Initial kernel scaffold; baseline (speedup 1.0000x reference)
import jax
import jax.numpy as jnp
from jax.experimental import pallas as pl


def kernel(state, edge_index, Wg, bg, W1, b1, W2, b2):
    raise NotImplementedError("write your pallas kernel here")


# speedup vs baseline: 42.7774x; 42.7774x over previous
import functools

import jax
import jax.numpy as jnp
from jax import lax
from jax.experimental import pallas as pl
from jax.experimental.pallas import tpu as pltpu
from jax.experimental.pallas import tpu_sc as plsc

N_NODES = 10000
NPAD = 10240
PADNODE = NPAD - 1
E_REAL = 320000
E_TOT = E_REAL + N_NODES
E_PAD = 2816 * 128
EROWS = 2816
RPT = EROWS // 32
G_PAD = 104
AFLAT = G_PAD * NPAD
A_PER_TILE = AFLAT // 16
DEG_PER_TILE = NPAD // 16
NB = 8
BLK = NPAD // NB

_mesh = plsc.VectorSubcoreMesh(core_axis_name="c", subcore_axis_name="s")


def _fast_rsqrt(d):
    xi = lax.bitcast_convert_type(d, jnp.int32)
    yi = jnp.int32(0x5F3759DF) - (xi >> 1)
    y = lax.bitcast_convert_type(yi, jnp.float32)
    for _ in range(3):
        y = y * (jnp.float32(1.5) - jnp.float32(0.5) * d * y * y)
    return y


@functools.partial(
    pl.kernel,
    out_type=jax.ShapeDtypeStruct((2 * NPAD,), jnp.float32),
    mesh=_mesh,
    scratch_types=[
        pltpu.VMEM((RPT, 128), jnp.int32),
        pltpu.VMEM((128,), jnp.float32),
        pltpu.VMEM((DEG_PER_TILE,), jnp.float32),
        pltpu.VMEM_SHARED((NPAD,), jnp.float32),
    ],
)
def _deg_kernel(dst_hbm, out_hbm, dst_v, ones_v, zeros_v, deg_sh):
    c = lax.axis_index("c")
    s = lax.axis_index("s")
    w = c * 16 + s

    def init_body(i, _):
        ones_v[pl.ds(i * 16, 16)] = jnp.ones((16,), jnp.float32)
        return 0

    lax.fori_loop(0, 128 // 16, init_body, 0)

    def zinit_body(i, _):
        zeros_v[pl.ds(i * 16, 16)] = jnp.zeros((16,), jnp.float32)
        return 0

    lax.fori_loop(0, DEG_PER_TILE // 16, zinit_body, 0)
    pltpu.sync_copy(zeros_v, deg_sh.at[pl.ds(s * DEG_PER_TILE, DEG_PER_TILE)])
    plsc.subcore_barrier()

    pltpu.sync_copy(dst_hbm.at[pl.ds(w * RPT, RPT)], dst_v)

    def edge_body(j, _):
        pltpu.sync_copy(ones_v, deg_sh.at[dst_v.at[j]], add=True)
        return 0

    lax.fori_loop(0, RPT, edge_body, 0)
    plsc.subcore_barrier()
    pltpu.sync_copy(
        deg_sh.at[pl.ds(s * DEG_PER_TILE, DEG_PER_TILE)],
        out_hbm.at[pl.ds(c * NPAD + s * DEG_PER_TILE, DEG_PER_TILE)],
    )


@functools.partial(
    pl.kernel,
    out_type=jax.ShapeDtypeStruct((2 * AFLAT,), jnp.float32),
    mesh=_mesh,
    compiler_params=pltpu.CompilerParams(needs_layout_passes=False),
    scratch_types=[
        pltpu.VMEM((8, 128), jnp.int32),
        pltpu.VMEM((8, 128), jnp.int32),
        pltpu.VMEM((8, 128), jnp.int32),
        pltpu.VMEM((8, 128), jnp.float32),
        pltpu.VMEM((NPAD,), jnp.float32),
        pltpu.VMEM((NPAD,), jnp.float32),
        pltpu.VMEM((A_PER_TILE // 32,), jnp.float32),
        pltpu.VMEM_SHARED((AFLAT,), jnp.float32),
    ],
)
def _a_kernel(src_hbm, dst_hbm, deg_hbm, a_out, src_v, dst_v, idx_v, val_v,
              dinv_v, tmp_v, zeros_v, a_sh):
    c = lax.axis_index("c")
    s = lax.axis_index("s")
    w = c * 16 + s
    zchunk = A_PER_TILE // 32

    def zinit_body(i, _):
        zeros_v[pl.ds(i * 16, 16)] = jnp.zeros((16,), jnp.float32)
        return 0

    lax.fori_loop(0, zchunk // 16, zinit_body, 0)

    def zcopy_body(t, _):
        pltpu.sync_copy(zeros_v, a_sh.at[pl.ds((s * 32 + t) * zchunk, zchunk)])
        return 0

    lax.fori_loop(0, 32, zcopy_body, 0)

    pltpu.sync_copy(deg_hbm.at[pl.ds(0, NPAD)], dinv_v)
    pltpu.sync_copy(deg_hbm.at[pl.ds(NPAD, NPAD)], tmp_v)

    def dinv_body(i, _):
        d = dinv_v[pl.ds(i * 16, 16)] + tmp_v[pl.ds(i * 16, 16)]
        y = _fast_rsqrt(d)
        dinv_v[pl.ds(i * 16, 16)] = jnp.where(d > 0.0, y, jnp.float32(0.0))
        return 0

    lax.fori_loop(0, NPAD // 16, dinv_body, 0)

    plsc.subcore_barrier()

    def chunk_body(q, _):
        base = w * RPT + q * 8
        pltpu.sync_copy(src_hbm.at[pl.ds(base, 8)], src_v)
        pltpu.sync_copy(dst_hbm.at[pl.ds(base, 8)], dst_v)

        def edge_body(j, _):
            def lane_body(l, _):
                dd = dst_v[j, pl.ds(l * 16, 16)]
                ss = src_v[j, pl.ds(l * 16, 16)]
                g = (dd * 5243) >> 19
                idx_v[j, pl.ds(l * 16, 16)] = g * NPAD + ss
                val_v[j, pl.ds(l * 16, 16)] = (
                    plsc.load_gather(dinv_v, [dd])
                    * plsc.load_gather(dinv_v, [ss])
                )
                return 0

            lax.fori_loop(0, 128 // 16, lane_body, 0)
            pltpu.sync_copy(val_v.at[j], a_sh.at[idx_v.at[j]], add=True)
            return 0

        lax.fori_loop(0, 8, edge_body, 0)
        return 0

    lax.fori_loop(0, RPT // 8, chunk_body, 0)
    plsc.subcore_barrier()
    pltpu.sync_copy(
        a_sh.at[pl.ds(s * A_PER_TILE, A_PER_TILE)],
        a_out.at[pl.ds(c * AFLAT + s * A_PER_TILE, A_PER_TILE)],
    )


def _tc_body(x_ref, a_ref, wg_ref, w1_ref, b1_ref, bg_ref, w2_ref, b2_ref,
             out_ref, pool, ssum):
    k = pl.program_id(0)

    @pl.when(k == 0)
    def _():
        pool[...] = jnp.zeros_like(pool)
        ssum[...] = jnp.zeros_like(ssum)

    x = x_ref[...]
    h = jnp.dot(x, wg_ref[...], preferred_element_type=jnp.float32)
    ab = a_ref[...]
    asum = ab[0:G_PAD, :] + ab[G_PAD:2 * G_PAD, :]
    pool[...] += jnp.dot(asum, h, preferred_element_type=jnp.float32)

    col = lax.broadcasted_iota(jnp.int32, (G_PAD, BLK), 1) + k * BLK
    grp = (col * 5243) >> 19
    row = lax.broadcasted_iota(jnp.int32, (G_PAD, BLK), 0)
    sel = jnp.where(grp == row, jnp.float32(1.0), jnp.float32(0.0))
    ssum[...] += jnp.dot(sel, x, preferred_element_type=jnp.float32)

    @pl.when(k == NB - 1)
    def _():
        w1a = w1_ref[0:128, :]
        w1b = w1_ref[128:256, :]
        bgw = jnp.dot(bg_ref[...], w1b, preferred_element_type=jnp.float32)
        vpre = (
            jnp.dot(ssum[...], w1a, preferred_element_type=jnp.float32)
            + jnp.dot(pool[...], w1b, preferred_element_type=jnp.float32)
            + b1_ref[...]
            + jnp.float32(100.0) * bgw
        )
        v = jnp.maximum(vpre, jnp.float32(0.0))
        out_ref[...] = (
            jnp.dot(v, w2_ref[...], preferred_element_type=jnp.float32)
            + b2_ref[...]
        )


def kernel(state, edge_index, Wg, bg, W1, b1, W2, b2):
    ei = edge_index.astype(jnp.int32)
    loops = jnp.arange(N_NODES, dtype=jnp.int32)
    pad = jnp.full((E_PAD - E_TOT,), PADNODE, dtype=jnp.int32)
    src2d = jnp.concatenate([ei[0], loops, pad]).reshape(EROWS, 128)
    dst2d = jnp.concatenate([ei[1], loops, pad]).reshape(EROWS, 128)

    deg = _deg_kernel(dst2d)
    a_parts = _a_kernel(src2d, dst2d, deg)
    a2 = a_parts.reshape(2 * G_PAD, NPAD)

    state_p = jnp.pad(state, ((0, NPAD - N_NODES), (0, 0)))

    out = pl.pallas_call(
        _tc_body,
        grid=(NB,),
        in_specs=[
            pl.BlockSpec((BLK, 128), lambda k: (k, 0)),
            pl.BlockSpec((2 * G_PAD, BLK), lambda k: (0, k)),
            pl.BlockSpec((128, 128), lambda k: (0, 0)),
            pl.BlockSpec((256, 128), lambda k: (0, 0)),
            pl.BlockSpec((1, 128), lambda k: (0, 0)),
            pl.BlockSpec((1, 128), lambda k: (0, 0)),
            pl.BlockSpec((128, 1), lambda k: (0, 0)),
            pl.BlockSpec((1, 1), lambda k: (0, 0)),
        ],
        out_specs=pl.BlockSpec((G_PAD, 1), lambda k: (0, 0)),
        out_shape=jax.ShapeDtypeStruct((G_PAD, 1), jnp.float32),
        scratch_shapes=[
            pltpu.VMEM((G_PAD, 128), jnp.float32),
            pltpu.VMEM((G_PAD, 128), jnp.float32),
        ],
    )(state_p, a2, Wg, W1, b1.reshape(1, 128), bg.reshape(1, 128), W2,
      b2.reshape(1, 1))
    return out[:100, 0]


# speedup vs baseline: 43.7319x; 1.0223x over previous
import functools

import jax
import jax.numpy as jnp
from jax import lax
from jax.experimental import pallas as pl
from jax.experimental.pallas import tpu as pltpu
from jax.experimental.pallas import tpu_sc as plsc

N_NODES = 10000
NPAD = 10240
PADNODE = NPAD - 1
E_REAL = 320000
E_TOT = E_REAL + N_NODES
E_PAD = 2816 * 128
EROWS = 2816
RPT = EROWS // 32
G_PAD = 104
AFLAT = G_PAD * NPAD
A_PER_TILE = AFLAT // 16
DEG_PER_TILE = NPAD // 16
NB = 8
BLK = NPAD // NB

_mesh = plsc.VectorSubcoreMesh(core_axis_name="c", subcore_axis_name="s")


def _fast_rsqrt(d):
    xi = lax.bitcast_convert_type(d, jnp.int32)
    yi = jnp.int32(0x5F3759DF) - (xi >> 1)
    y = lax.bitcast_convert_type(yi, jnp.float32)
    for _ in range(3):
        y = y * (jnp.float32(1.5) - jnp.float32(0.5) * d * y * y)
    return y


@functools.partial(
    pl.kernel,
    out_type=jax.ShapeDtypeStruct((2 * NPAD,), jnp.float32),
    mesh=_mesh,
    scratch_types=[
        pltpu.VMEM((RPT, 128), jnp.int32),
        pltpu.VMEM((128,), jnp.float32),
        pltpu.VMEM((DEG_PER_TILE,), jnp.float32),
        pltpu.VMEM_SHARED((NPAD,), jnp.float32),
        pltpu.SemaphoreType.DMA,
    ],
)
def _deg_kernel(dst_hbm, out_hbm, dst_v, ones_v, zeros_v, deg_sh, sem):
    c = lax.axis_index("c")
    s = lax.axis_index("s")
    w = c * 16 + s

    def init_body(i, _):
        ones_v[pl.ds(i * 16, 16)] = jnp.ones((16,), jnp.float32)
        return 0

    lax.fori_loop(0, 128 // 16, init_body, 0)

    def zinit_body(i, _):
        zeros_v[pl.ds(i * 16, 16)] = jnp.zeros((16,), jnp.float32)
        return 0

    lax.fori_loop(0, DEG_PER_TILE // 16, zinit_body, 0)
    pltpu.sync_copy(zeros_v, deg_sh.at[pl.ds(s * DEG_PER_TILE, DEG_PER_TILE)])
    plsc.subcore_barrier()

    pltpu.sync_copy(dst_hbm.at[pl.ds(w * RPT, RPT)], dst_v)

    def fire_body(j, _):
        pltpu.async_copy(ones_v, deg_sh.at[dst_v.at[j]], sem, add=True)
        return 0

    lax.fori_loop(0, RPT, fire_body, 0)

    def drain_body(j, _):
        pltpu.make_async_copy(ones_v, deg_sh.at[dst_v.at[j]], sem).wait()
        return 0

    lax.fori_loop(0, RPT, drain_body, 0)
    plsc.subcore_barrier()
    pltpu.sync_copy(
        deg_sh.at[pl.ds(s * DEG_PER_TILE, DEG_PER_TILE)],
        out_hbm.at[pl.ds(c * NPAD + s * DEG_PER_TILE, DEG_PER_TILE)],
    )


@functools.partial(
    pl.kernel,
    out_type=jax.ShapeDtypeStruct((2 * AFLAT,), jnp.float32),
    mesh=_mesh,
    compiler_params=pltpu.CompilerParams(needs_layout_passes=False),
    scratch_types=[
        pltpu.VMEM((8, 128), jnp.int32),
        pltpu.VMEM((8, 128), jnp.int32),
        pltpu.VMEM((RPT, 128), jnp.int32),
        pltpu.VMEM((RPT, 128), jnp.float32),
        pltpu.VMEM((NPAD,), jnp.float32),
        pltpu.VMEM((NPAD,), jnp.float32),
        pltpu.VMEM((A_PER_TILE // 32,), jnp.float32),
        pltpu.VMEM_SHARED((AFLAT,), jnp.float32),
        pltpu.SemaphoreType.DMA,
    ],
)
def _a_kernel(src_hbm, dst_hbm, deg_hbm, a_out, src_v, dst_v, idx_v, val_v,
              dinv_v, tmp_v, zeros_v, a_sh, sem):
    c = lax.axis_index("c")
    s = lax.axis_index("s")
    w = c * 16 + s
    zchunk = A_PER_TILE // 32

    def zinit_body(i, _):
        zeros_v[pl.ds(i * 16, 16)] = jnp.zeros((16,), jnp.float32)
        return 0

    lax.fori_loop(0, zchunk // 16, zinit_body, 0)

    def zcopy_body(t, _):
        pltpu.async_copy(zeros_v, a_sh.at[pl.ds((s * 32 + t) * zchunk, zchunk)],
                         sem)
        return 0

    lax.fori_loop(0, 32, zcopy_body, 0)

    pltpu.sync_copy(deg_hbm.at[pl.ds(0, NPAD)], dinv_v)
    pltpu.sync_copy(deg_hbm.at[pl.ds(NPAD, NPAD)], tmp_v)

    def dinv_body(i, _):
        d = dinv_v[pl.ds(i * 16, 16)] + tmp_v[pl.ds(i * 16, 16)]
        y = _fast_rsqrt(d)
        dinv_v[pl.ds(i * 16, 16)] = jnp.where(d > 0.0, y, jnp.float32(0.0))
        return 0

    lax.fori_loop(0, NPAD // 16, dinv_body, 0)

    def zdrain_body(t, _):
        pltpu.make_async_copy(
            zeros_v, a_sh.at[pl.ds((s * 32 + t) * zchunk, zchunk)], sem
        ).wait()
        return 0

    lax.fori_loop(0, 32, zdrain_body, 0)
    plsc.subcore_barrier()

    def chunk_body(q, _):
        base = w * RPT + q * 8
        pltpu.sync_copy(src_hbm.at[pl.ds(base, 8)], src_v)
        pltpu.sync_copy(dst_hbm.at[pl.ds(base, 8)], dst_v)

        def edge_body(j, _):
            r = q * 8 + j

            def lane_body(l, _):
                dd = dst_v[j, pl.ds(l * 16, 16)]
                ss = src_v[j, pl.ds(l * 16, 16)]
                g = (dd * 5243) >> 19
                idx_v[r, pl.ds(l * 16, 16)] = g * NPAD + ss
                val_v[r, pl.ds(l * 16, 16)] = (
                    plsc.load_gather(dinv_v, [dd])
                    * plsc.load_gather(dinv_v, [ss])
                )
                return 0

            lax.fori_loop(0, 128 // 16, lane_body, 0)
            pltpu.async_copy(val_v.at[r], a_sh.at[idx_v.at[r]], sem, add=True)
            return 0

        lax.fori_loop(0, 8, edge_body, 0)
        return 0

    lax.fori_loop(0, RPT // 8, chunk_body, 0)

    def sdrain_body(j, _):
        pltpu.make_async_copy(val_v.at[j], a_sh.at[idx_v.at[j]], sem).wait()
        return 0

    lax.fori_loop(0, RPT, sdrain_body, 0)
    plsc.subcore_barrier()
    pltpu.sync_copy(
        a_sh.at[pl.ds(s * A_PER_TILE, A_PER_TILE)],
        a_out.at[pl.ds(c * AFLAT + s * A_PER_TILE, A_PER_TILE)],
    )


def _tc_body(x_ref, a_ref, wg_ref, w1_ref, b1_ref, bg_ref, w2_ref, b2_ref,
             out_ref, pool, ssum):
    k = pl.program_id(0)

    @pl.when(k == 0)
    def _():
        pool[...] = jnp.zeros_like(pool)
        ssum[...] = jnp.zeros_like(ssum)

    x = x_ref[...]
    h = jnp.dot(x, wg_ref[...], preferred_element_type=jnp.float32)
    ab = a_ref[...]
    asum = ab[0:G_PAD, :] + ab[G_PAD:2 * G_PAD, :]
    pool[...] += jnp.dot(asum, h, preferred_element_type=jnp.float32)

    col = lax.broadcasted_iota(jnp.int32, (G_PAD, BLK), 1) + k * BLK
    grp = (col * 5243) >> 19
    row = lax.broadcasted_iota(jnp.int32, (G_PAD, BLK), 0)
    sel = jnp.where(grp == row, jnp.float32(1.0), jnp.float32(0.0))
    ssum[...] += jnp.dot(sel, x, preferred_element_type=jnp.float32)

    @pl.when(k == NB - 1)
    def _():
        w1a = w1_ref[0:128, :]
        w1b = w1_ref[128:256, :]
        bgw = jnp.dot(bg_ref[...], w1b, preferred_element_type=jnp.float32)
        vpre = (
            jnp.dot(ssum[...], w1a, preferred_element_type=jnp.float32)
            + jnp.dot(pool[...], w1b, preferred_element_type=jnp.float32)
            + b1_ref[...]
            + jnp.float32(100.0) * bgw
        )
        v = jnp.maximum(vpre, jnp.float32(0.0))
        out_ref[...] = (
            jnp.dot(v, w2_ref[...], preferred_element_type=jnp.float32)
            + b2_ref[...]
        )


def kernel(state, edge_index, Wg, bg, W1, b1, W2, b2):
    ei = edge_index.astype(jnp.int32)
    loops = jnp.arange(N_NODES, dtype=jnp.int32)
    pad = jnp.full((E_PAD - E_TOT,), PADNODE, dtype=jnp.int32)
    src2d = jnp.concatenate([ei[0], loops, pad]).reshape(EROWS, 128)
    dst2d = jnp.concatenate([ei[1], loops, pad]).reshape(EROWS, 128)

    deg = _deg_kernel(dst2d)
    a_parts = _a_kernel(src2d, dst2d, deg)
    a2 = a_parts.reshape(2 * G_PAD, NPAD)

    state_p = jnp.pad(state, ((0, NPAD - N_NODES), (0, 0)))

    out = pl.pallas_call(
        _tc_body,
        grid=(NB,),
        in_specs=[
            pl.BlockSpec((BLK, 128), lambda k: (k, 0)),
            pl.BlockSpec((2 * G_PAD, BLK), lambda k: (0, k)),
            pl.BlockSpec((128, 128), lambda k: (0, 0)),
            pl.BlockSpec((256, 128), lambda k: (0, 0)),
            pl.BlockSpec((1, 128), lambda k: (0, 0)),
            pl.BlockSpec((1, 128), lambda k: (0, 0)),
            pl.BlockSpec((128, 1), lambda k: (0, 0)),
            pl.BlockSpec((1, 1), lambda k: (0, 0)),
        ],
        out_specs=pl.BlockSpec((G_PAD, 1), lambda k: (0, 0)),
        out_shape=jax.ShapeDtypeStruct((G_PAD, 1), jnp.float32),
        scratch_shapes=[
            pltpu.VMEM((G_PAD, 128), jnp.float32),
            pltpu.VMEM((G_PAD, 128), jnp.float32),
        ],
    )(state_p, a2, Wg, W1, b1.reshape(1, 128), bg.reshape(1, 128), W2,
      b2.reshape(1, 1))
    return out[:100, 0]


# speedup vs baseline: 71.4406x; 1.6336x over previous
import functools

import jax
import jax.numpy as jnp
from jax import lax
from jax.experimental import pallas as pl
from jax.experimental.pallas import tpu as pltpu
from jax.experimental.pallas import tpu_sc as plsc

N_NODES = 10000
NPAD = 10240
E_REAL = 320000
EPT = E_REAL // 32
EGROUPS = EPT // 16
DEGROWS = 79
DUMP = NPAD
SLPT = NPAD // 32
SLG = SLPT // 16
TOTG = EGROUPS + SLG + 3
ROWS = TOTG // 8
GSTRIDE = 128
ATFLAT = NPAD * GSTRIDE
AT_PER_TILE = ATFLAT // 16
ATOUT = N_NODES * GSTRIDE
ATOUT_PER_TILE = ATOUT // 16
DEG_PER_TILE = NPAD // 16
ECHUNK = 2000
NB = 5
BLK = N_NODES // NB

_mesh = plsc.VectorSubcoreMesh(core_axis_name="c", subcore_axis_name="s")


def _fast_rsqrt(d):
    xi = lax.bitcast_convert_type(d, jnp.int32)
    yi = jnp.int32(0x5F3759DF) - (xi >> 1)
    y = lax.bitcast_convert_type(yi, jnp.float32)
    for _ in range(3):
        y = y * (jnp.float32(1.5) - jnp.float32(0.5) * d * y * y)
    return y


@functools.partial(
    pl.kernel,
    out_type=jax.ShapeDtypeStruct((2 * NPAD,), jnp.float32),
    mesh=_mesh,
    scratch_types=[
        pltpu.VMEM((DEGROWS, 128), jnp.int32),
        pltpu.VMEM((128,), jnp.float32),
        pltpu.VMEM((DEG_PER_TILE + 8,), jnp.float32),
        pltpu.VMEM_SHARED((NPAD + 128,), jnp.float32),
        pltpu.SemaphoreType.DMA,
    ],
)
def _deg_kernel(dst_hbm, out_hbm, idx_v, ones_v, zeros_v, deg_sh, sem):
    c = lax.axis_index("c")
    s = lax.axis_index("s")
    w = c * 16 + s
    base = w * EPT

    def init_body(i, _):
        ones_v[pl.ds(i * 16, 16)] = jnp.ones((16,), jnp.float32)
        return 0

    lax.fori_loop(0, 128 // 16, init_body, 0)

    def zinit_body(i, _):
        zeros_v[pl.ds(i * 16, 16)] = jnp.zeros((16,), jnp.float32)
        return 0

    lax.fori_loop(0, (DEG_PER_TILE + 8) // 16, zinit_body, 0)
    pltpu.sync_copy(zeros_v, deg_sh.at[pl.ds(s * (DEG_PER_TILE + 8),
                                             DEG_PER_TILE + 8)])

    def fill_body(l, _):
        idx_v[DEGROWS - 1, pl.ds(l * 16, 16)] = jnp.full((16,), DUMP,
                                                         jnp.int32)
        return 0

    lax.fori_loop(0, 8, fill_body, 0)

    def load_body(r, _):
        pltpu.async_copy(dst_hbm.at[pl.ds(base + r * 128, 128)],
                         idx_v.at[r], sem)
        return 0

    lax.fori_loop(0, DEGROWS - 1, load_body, 0)
    pltpu.async_copy(dst_hbm.at[pl.ds(base + (DEGROWS - 1) * 128, 16)],
                     idx_v.at[DEGROWS - 1, pl.ds(0, 16)], sem)

    def ldrain_body(r, _):
        pltpu.make_async_copy(dst_hbm.at[pl.ds(base + r * 128, 128)],
                              idx_v.at[r], sem).wait()
        return 0

    lax.fori_loop(0, DEGROWS - 1, ldrain_body, 0)
    pltpu.make_async_copy(dst_hbm.at[pl.ds(base + (DEGROWS - 1) * 128, 16)],
                          idx_v.at[DEGROWS - 1, pl.ds(0, 16)], sem).wait()
    plsc.subcore_barrier()

    def fire_body(r, _):
        pltpu.async_copy(ones_v, deg_sh.at[idx_v.at[r]], sem, add=True)
        return 0

    lax.fori_loop(0, DEGROWS, fire_body, 0)

    def drain_body(r, _):
        pltpu.make_async_copy(ones_v, deg_sh.at[idx_v.at[r]], sem).wait()
        return 0

    lax.fori_loop(0, DEGROWS, drain_body, 0)
    plsc.subcore_barrier()
    pltpu.sync_copy(
        deg_sh.at[pl.ds(s * DEG_PER_TILE, DEG_PER_TILE)],
        out_hbm.at[pl.ds(c * NPAD + s * DEG_PER_TILE, DEG_PER_TILE)],
    )


@functools.partial(
    pl.kernel,
    out_type=jax.ShapeDtypeStruct((2 * ATOUT,), jnp.float32),
    mesh=_mesh,
    compiler_params=pltpu.CompilerParams(needs_layout_passes=False),
    scratch_types=[
        pltpu.VMEM((ECHUNK,), jnp.int32),
        pltpu.VMEM((ECHUNK,), jnp.int32),
        pltpu.VMEM((ROWS, 128), jnp.int32),
        pltpu.VMEM((ROWS, 128), jnp.float32),
        pltpu.VMEM((NPAD,), jnp.float32),
        pltpu.VMEM((NPAD,), jnp.float32),
        pltpu.VMEM((AT_PER_TILE // 64,), jnp.float32),
        pltpu.VMEM_SHARED((ATFLAT,), jnp.float32),
        pltpu.SemaphoreType.DMA,
    ],
)
def _a_kernel(src_hbm, dst_hbm, deg_hbm, a_out, src_v, dst_v, idx_v, val_v,
              dinv_v, tmp_v, zeros_v, a_sh, sem):
    c = lax.axis_index("c")
    s = lax.axis_index("s")
    w = c * 16 + s
    base = w * EPT
    zchunk = AT_PER_TILE // 64

    def zinit_body(i, _):
        zeros_v[pl.ds(i * 16, 16)] = jnp.zeros((16,), jnp.float32)
        return 0

    lax.fori_loop(0, zchunk // 16, zinit_body, 0)

    def zcopy_body(t, _):
        pltpu.async_copy(zeros_v, a_sh.at[pl.ds((s * 64 + t) * zchunk,
                                                zchunk)], sem)
        return 0

    lax.fori_loop(0, 64, zcopy_body, 0)

    pltpu.sync_copy(deg_hbm.at[pl.ds(0, NPAD)], dinv_v)
    pltpu.sync_copy(deg_hbm.at[pl.ds(NPAD, NPAD)], tmp_v)

    def dinv_body(i, _):
        d = (dinv_v[pl.ds(i * 16, 16)] + tmp_v[pl.ds(i * 16, 16)]
             + jnp.float32(1.0))
        dinv_v[pl.ds(i * 16, 16)] = _fast_rsqrt(d)
        return 0

    lax.fori_loop(0, NPAD // 16, dinv_body, 0)

    def zdrain_body(t, _):
        pltpu.make_async_copy(
            zeros_v, a_sh.at[pl.ds((s * 64 + t) * zchunk, zchunk)], sem
        ).wait()
        return 0

    lax.fori_loop(0, 64, zdrain_body, 0)
    plsc.subcore_barrier()

    def chunk_body(q, _):
        pltpu.sync_copy(src_hbm.at[pl.ds(base + q * ECHUNK, ECHUNK)], src_v)
        pltpu.sync_copy(dst_hbm.at[pl.ds(base + q * ECHUNK, ECHUNK)], dst_v)

        def grp_body(t, _):
            gidx = q * 125 + t
            r = gidx >> 3
            l = gidx & 7
            dd = dst_v[pl.ds(t * 16, 16)]
            ss = src_v[pl.ds(t * 16, 16)]
            g = (dd * 5243) >> 19
            idx_v[r, pl.ds(l * 16, 16)] = (ss << 7) + g
            val_v[r, pl.ds(l * 16, 16)] = (
                plsc.load_gather(dinv_v, [dd])
                * plsc.load_gather(dinv_v, [ss])
            )
            return 0

        lax.fori_loop(0, 125, grp_body, 0)
        return 0

    lax.fori_loop(0, 5, chunk_body, 0)

    lane = jnp.arange(16, dtype=jnp.int32)

    def sl_body(t, _):
        gidx = EGROUPS + t
        r = gidx >> 3
        l = gidx & 7
        n = w * SLPT + t * 16 + lane
        dv = dinv_v[pl.ds(w * SLPT + t * 16, 16)]
        g = (n * 5243) >> 19
        idx_v[r, pl.ds(l * 16, 16)] = (n << 7) + g
        val_v[r, pl.ds(l * 16, 16)] = jnp.where(
            n < N_NODES, dv * dv, jnp.float32(0.0))
        return 0

    lax.fori_loop(0, SLG, sl_body, 0)

    def fill_body(t, _):
        gidx = EGROUPS + SLG + t
        r = gidx >> 3
        l = gidx & 7
        idx_v[r, pl.ds(l * 16, 16)] = jnp.zeros((16,), jnp.int32)
        val_v[r, pl.ds(l * 16, 16)] = jnp.zeros((16,), jnp.float32)
        return 0

    lax.fori_loop(0, 3, fill_body, 0)

    def fire_body(r, _):
        pltpu.async_copy(val_v.at[r], a_sh.at[idx_v.at[r]], sem, add=True)
        return 0

    lax.fori_loop(0, ROWS, fire_body, 0)

    def drain_body(r, _):
        pltpu.make_async_copy(val_v.at[r], a_sh.at[idx_v.at[r]], sem).wait()
        return 0

    lax.fori_loop(0, ROWS, drain_body, 0)
    plsc.subcore_barrier()
    pltpu.sync_copy(
        a_sh.at[pl.ds(s * ATOUT_PER_TILE, ATOUT_PER_TILE)],
        a_out.at[pl.ds(c * ATOUT + s * ATOUT_PER_TILE, ATOUT_PER_TILE)],
    )


def _tc_body(x_ref, at_ref, wg_ref, w1_ref, b1_ref, bg_ref, w2_ref, b2_ref,
             out_ref, pool, ssum):
    k = pl.program_id(0)

    @pl.when(k == 0)
    def _():
        pool[...] = jnp.zeros_like(pool)
        ssum[...] = jnp.zeros_like(ssum)

    x = x_ref[...]
    h = jnp.dot(x, wg_ref[...], preferred_element_type=jnp.float32)
    at = at_ref[...]
    asum = at[0] + at[1]
    pool[...] += lax.dot_general(
        asum, h, (((0,), (0,)), ((), ())),
        preferred_element_type=jnp.float32)

    col = lax.broadcasted_iota(jnp.int32, (128, BLK), 1) + k * BLK
    grp = (col * 5243) >> 19
    row = lax.broadcasted_iota(jnp.int32, (128, BLK), 0)
    sel = jnp.where(grp == row, jnp.float32(1.0), jnp.float32(0.0))
    ssum[...] += jnp.dot(sel, x, preferred_element_type=jnp.float32)

    @pl.when(k == NB - 1)
    def _():
        w1a = w1_ref[0:128, :]
        w1b = w1_ref[128:256, :]
        bgw = jnp.dot(bg_ref[...], w1b, preferred_element_type=jnp.float32)
        vpre = (
            jnp.dot(ssum[...], w1a, preferred_element_type=jnp.float32)
            + jnp.dot(pool[...], w1b, preferred_element_type=jnp.float32)
            + b1_ref[...]
            + jnp.float32(100.0) * bgw
        )
        v = jnp.maximum(vpre, jnp.float32(0.0))
        out_ref[...] = (
            jnp.dot(v, w2_ref[...], preferred_element_type=jnp.float32)
            + b2_ref[...]
        )


def kernel(state, edge_index, Wg, bg, W1, b1, W2, b2):
    ei = edge_index.astype(jnp.int32)
    src = ei[0]
    dst = ei[1]

    deg = _deg_kernel(dst)
    at_parts = _a_kernel(src, dst, deg)
    at3 = at_parts.reshape(2, N_NODES, GSTRIDE)

    out = pl.pallas_call(
        _tc_body,
        grid=(NB,),
        in_specs=[
            pl.BlockSpec((BLK, 128), lambda k: (k, 0)),
            pl.BlockSpec((2, BLK, 128), lambda k: (0, k, 0)),
            pl.BlockSpec((128, 128), lambda k: (0, 0)),
            pl.BlockSpec((256, 128), lambda k: (0, 0)),
            pl.BlockSpec((1, 128), lambda k: (0, 0)),
            pl.BlockSpec((1, 128), lambda k: (0, 0)),
            pl.BlockSpec((128, 1), lambda k: (0, 0)),
            pl.BlockSpec((1, 1), lambda k: (0, 0)),
        ],
        out_specs=pl.BlockSpec((128, 1), lambda k: (0, 0)),
        out_shape=jax.ShapeDtypeStruct((128, 1), jnp.float32),
        scratch_shapes=[
            pltpu.VMEM((128, 128), jnp.float32),
            pltpu.VMEM((128, 128), jnp.float32),
        ],
    )(state, at3, Wg, W1, b1.reshape(1, 128), bg.reshape(1, 128), W2,
      b2.reshape(1, 1))
    return out[:100, 0]


# speedup vs baseline: 88.2617x; 1.2355x over previous
import functools

import jax
import jax.numpy as jnp
from jax import lax
from jax.experimental import pallas as pl
from jax.experimental.pallas import tpu as pltpu
from jax.experimental.pallas import tpu_sc as plsc

N_NODES = 10000
NPAD = 10240
E_REAL = 320000
EPT = E_REAL // 32
EGROUPS = EPT // 16
ECHUNK = 2000
DOFF = E_REAL
DEGROWS = 79
DUMP = NPAD
SLPT = NPAD // 32
SLG = SLPT // 16
ROWS = 82
GSTRIDE = 128
ATFLAT = NPAD * GSTRIDE
AT_PER_TILE = ATFLAT // 16
ATOUT = N_NODES * GSTRIDE
ATOUT_PER_TILE = ATOUT // 16
DEG_PER_TILE = NPAD // 16
NB = 5
BLK = N_NODES // NB

_mesh = plsc.VectorSubcoreMesh(core_axis_name="c", subcore_axis_name="s")


def _fast_rsqrt(d):
    xi = lax.bitcast_convert_type(d, jnp.int32)
    yi = jnp.int32(0x5F3759DF) - (xi >> 1)
    y = lax.bitcast_convert_type(yi, jnp.float32)
    for _ in range(3):
        y = y * (jnp.float32(1.5) - jnp.float32(0.5) * d * y * y)
    return y


@functools.partial(
    pl.kernel,
    out_type=jax.ShapeDtypeStruct((2 * NPAD,), jnp.float32),
    mesh=_mesh,
    compiler_params=pltpu.CompilerParams(needs_layout_passes=False),
    scratch_types=[
        pltpu.VMEM((ECHUNK,), jnp.int32),
        pltpu.VMEM((DEGROWS, 128), jnp.int32),
        pltpu.VMEM((128,), jnp.float32),
        pltpu.VMEM((DEG_PER_TILE + 8,), jnp.float32),
        pltpu.VMEM_SHARED((NPAD + 128,), jnp.float32),
        pltpu.SemaphoreType.DMA,
    ],
)
def _deg_kernel(bc_hbm, out_hbm, dst_v, idx_v, ones_v, zeros_v, deg_sh, sem):
    c = lax.axis_index("c")
    s = lax.axis_index("s")
    w = c * 16 + s
    base = DOFF + w * EPT

    def zinit_body(i, _):
        zeros_v[pl.ds(i * 16, 16)] = jnp.zeros((16,), jnp.float32)
        return 0

    lax.fori_loop(0, (DEG_PER_TILE + 8) // 16, zinit_body, 0)
    pltpu.sync_copy(zeros_v, deg_sh.at[pl.ds(s * (DEG_PER_TILE + 8),
                                             DEG_PER_TILE + 8)])

    def ones_body(l, _):
        ones_v[pl.ds(l * 16, 16)] = jnp.ones((16,), jnp.float32)
        return 0

    lax.fori_loop(0, 8, ones_body, 0)

    for g0 in range(625, 632):
        idx_v[g0 >> 3, pl.ds((g0 & 7) * 16, 16)] = jnp.full((16,), DUMP,
                                                            jnp.int32)

    for q in range(5):
        pltpu.sync_copy(bc_hbm.at[pl.ds(base + q * ECHUNK, ECHUNK)], dst_v)

        def grp_body(t2, _):
            for u in range(5):
                t = t2 * 5 + u
                gidx = q * 125 + t
                dd = dst_v[pl.ds(t * 16, 16)]
                idx_v[gidx >> 3, pl.ds((gidx & 7) * 16, 16)] = dd
            return 0

        lax.fori_loop(0, 25, grp_body, 0)

    plsc.subcore_barrier()

    def fire_body(r, _):
        pltpu.async_copy(ones_v, deg_sh.at[idx_v.at[r]], sem, add=True)
        return 0

    lax.fori_loop(0, DEGROWS, fire_body, 0)

    def drain_body(r, _):
        pltpu.make_async_copy(ones_v, deg_sh.at[idx_v.at[r]], sem).wait()
        return 0

    lax.fori_loop(0, DEGROWS, drain_body, 0)
    plsc.subcore_barrier()
    pltpu.sync_copy(
        deg_sh.at[pl.ds(s * DEG_PER_TILE, DEG_PER_TILE)],
        out_hbm.at[pl.ds(c * NPAD + s * DEG_PER_TILE, DEG_PER_TILE)],
    )


@functools.partial(
    pl.kernel,
    out_type=jax.ShapeDtypeStruct((2 * ATOUT,), jnp.float32),
    mesh=_mesh,
    compiler_params=pltpu.CompilerParams(needs_layout_passes=False),
    scratch_types=[
        pltpu.VMEM((ECHUNK,), jnp.int32),
        pltpu.VMEM((ECHUNK,), jnp.int32),
        pltpu.VMEM((ROWS, 128), jnp.int32),
        pltpu.VMEM((ROWS, 128), jnp.float32),
        pltpu.VMEM((NPAD,), jnp.float32),
        pltpu.VMEM((DEG_PER_TILE,), jnp.float32),
        pltpu.VMEM((AT_PER_TILE // 64,), jnp.float32),
        pltpu.VMEM_SHARED((ATFLAT,), jnp.float32),
        pltpu.VMEM_SHARED((NPAD,), jnp.float32),
        pltpu.SemaphoreType.DMA,
    ],
)
def _a_kernel(bc_hbm, deg_hbm, a_out, src_v, dst_v, idx_v, val_v,
              dinv_v, dslc_v, zeros_v, a_sh, dinv_sh, sem):
    c = lax.axis_index("c")
    s = lax.axis_index("s")
    w = c * 16 + s
    base = w * EPT
    zchunk = AT_PER_TILE // 64
    lane1 = jnp.arange(16, dtype=jnp.int32)

    def zinit_body(i, _):
        zeros_v[pl.ds(i * 16, 16)] = jnp.zeros((16,), jnp.float32)
        return 0

    lax.fori_loop(0, zchunk // 16, zinit_body, 0)

    def zcopy_body(t, _):
        pltpu.async_copy(zeros_v, a_sh.at[pl.ds((s * 64 + t) * zchunk,
                                                zchunk)], sem)
        return 0

    lax.fori_loop(0, 64, zcopy_body, 0)

    pltpu.sync_copy(deg_hbm.at[pl.ds(s * DEG_PER_TILE, DEG_PER_TILE)],
                    dslc_v)
    pltpu.sync_copy(deg_hbm.at[pl.ds(NPAD + s * DEG_PER_TILE, DEG_PER_TILE)],
                    dinv_v.at[pl.ds(0, DEG_PER_TILE)])

    def dinv_body(i, _):
        d = (dslc_v[pl.ds(i * 16, 16)] + dinv_v[pl.ds(i * 16, 16)]
             + jnp.float32(1.0))
        dslc_v[pl.ds(i * 16, 16)] = _fast_rsqrt(d)
        return 0

    lax.fori_loop(0, DEG_PER_TILE // 16, dinv_body, 0)
    pltpu.sync_copy(dslc_v, dinv_sh.at[pl.ds(s * DEG_PER_TILE,
                                             DEG_PER_TILE)])

    def zdrain_body(t, _):
        pltpu.make_async_copy(
            zeros_v, a_sh.at[pl.ds((s * 64 + t) * zchunk, zchunk)], sem
        ).wait()
        return 0

    lax.fori_loop(0, 64, zdrain_body, 0)
    plsc.subcore_barrier()
    pltpu.sync_copy(dinv_sh, dinv_v)

    for q in range(5):
        pltpu.sync_copy(bc_hbm.at[pl.ds(base + q * ECHUNK, ECHUNK)], src_v)
        pltpu.sync_copy(bc_hbm.at[pl.ds(DOFF + base + q * ECHUNK, ECHUNK)],
                        dst_v)

        def grp_body(t2, _):
            for u in range(5):
                t = t2 * 5 + u
                gidx = q * 125 + t
                dd = dst_v[pl.ds(t * 16, 16)]
                ss = src_v[pl.ds(t * 16, 16)]
                g = (dd * 5243) >> 19
                idx_v[gidx >> 3, pl.ds((gidx & 7) * 16, 16)] = (ss << 7) + g
                val_v[gidx >> 3, pl.ds((gidx & 7) * 16, 16)] = (
                    plsc.load_gather(dinv_v, [dd])
                    * plsc.load_gather(dinv_v, [ss])
                )
            return 0

        lax.fori_loop(0, 25, grp_body, 0)

    for g0 in range(625, 632):
        idx_v[g0 >> 3, pl.ds((g0 & 7) * 16, 16)] = jnp.zeros((16,),
                                                             jnp.int32)
        val_v[g0 >> 3, pl.ds((g0 & 7) * 16, 16)] = jnp.zeros((16,),
                                                             jnp.float32)

    for t in range(SLG):
        gidx = 632 + t
        n = w * SLPT + t * 16 + lane1
        dv = plsc.load_gather(dinv_v, [n])
        g = (n * 5243) >> 19
        idx_v[gidx >> 3, pl.ds((gidx & 7) * 16, 16)] = (n << 7) + g
        val_v[gidx >> 3, pl.ds((gidx & 7) * 16, 16)] = jnp.where(
            n < N_NODES, dv * dv, jnp.float32(0.0))

    for g0 in range(652, 656):
        idx_v[g0 >> 3, pl.ds((g0 & 7) * 16, 16)] = jnp.zeros((16,),
                                                             jnp.int32)
        val_v[g0 >> 3, pl.ds((g0 & 7) * 16, 16)] = jnp.zeros((16,),
                                                             jnp.float32)

    def fire_body(r, _):
        pltpu.async_copy(val_v.at[r], a_sh.at[idx_v.at[r]], sem, add=True)
        return 0

    lax.fori_loop(0, ROWS, fire_body, 0)

    def drain_body(r, _):
        pltpu.make_async_copy(val_v.at[r], a_sh.at[idx_v.at[r]], sem).wait()
        return 0

    lax.fori_loop(0, ROWS, drain_body, 0)
    plsc.subcore_barrier()
    pltpu.sync_copy(
        a_sh.at[pl.ds(s * ATOUT_PER_TILE, ATOUT_PER_TILE)],
        a_out.at[pl.ds(c * ATOUT + s * ATOUT_PER_TILE, ATOUT_PER_TILE)],
    )


def _tc_body(x_ref, at_ref, wg_ref, w1_ref, b1_ref, bg_ref, w2_ref, b2_ref,
             out_ref, pool, ssum):
    k = pl.program_id(0)

    @pl.when(k == 0)
    def _():
        pool[...] = jnp.zeros_like(pool)
        ssum[...] = jnp.zeros_like(ssum)

    x = x_ref[...]
    h = jnp.dot(x, wg_ref[...], preferred_element_type=jnp.float32)
    at = at_ref[...]
    asum = at[0] + at[1]
    pool[...] += lax.dot_general(
        asum, h, (((0,), (0,)), ((), ())),
        preferred_element_type=jnp.float32)

    col = lax.broadcasted_iota(jnp.int32, (128, BLK), 1) + k * BLK
    grp = (col * 5243) >> 19
    row = lax.broadcasted_iota(jnp.int32, (128, BLK), 0)
    sel = jnp.where(grp == row, jnp.float32(1.0), jnp.float32(0.0))
    ssum[...] += jnp.dot(sel, x, preferred_element_type=jnp.float32)

    @pl.when(k == NB - 1)
    def _():
        w1a = w1_ref[0:128, :]
        w1b = w1_ref[128:256, :]
        bgw = jnp.dot(bg_ref[...], w1b, preferred_element_type=jnp.float32)
        vpre = (
            jnp.dot(ssum[...], w1a, preferred_element_type=jnp.float32)
            + jnp.dot(pool[...], w1b, preferred_element_type=jnp.float32)
            + b1_ref[...]
            + jnp.float32(100.0) * bgw
        )
        v = jnp.maximum(vpre, jnp.float32(0.0))
        out_ref[...] = (
            jnp.dot(v, w2_ref[...], preferred_element_type=jnp.float32)
            + b2_ref[...]
        )


def kernel(state, edge_index, Wg, bg, W1, b1, W2, b2):
    bc = edge_index.astype(jnp.int32).reshape(-1)

    deg = _deg_kernel(bc)
    at_parts = _a_kernel(bc, deg)
    at3 = at_parts.reshape(2, N_NODES, GSTRIDE)

    out = pl.pallas_call(
        _tc_body,
        grid=(NB,),
        in_specs=[
            pl.BlockSpec((BLK, 128), lambda k: (k, 0)),
            pl.BlockSpec((2, BLK, 128), lambda k: (0, k, 0)),
            pl.BlockSpec((128, 128), lambda k: (0, 0)),
            pl.BlockSpec((256, 128), lambda k: (0, 0)),
            pl.BlockSpec((1, 128), lambda k: (0, 0)),
            pl.BlockSpec((1, 128), lambda k: (0, 0)),
            pl.BlockSpec((128, 1), lambda k: (0, 0)),
            pl.BlockSpec((1, 1), lambda k: (0, 0)),
        ],
        out_specs=pl.BlockSpec((128, 1), lambda k: (0, 0)),
        out_shape=jax.ShapeDtypeStruct((128, 1), jnp.float32),
        scratch_shapes=[
            pltpu.VMEM((128, 128), jnp.float32),
            pltpu.VMEM((128, 128), jnp.float32),
        ],
    )(state, at3, Wg, W1, b1.reshape(1, 128), bg.reshape(1, 128), W2,
      b2.reshape(1, 1))
    return out[:100, 0]


# speedup vs baseline: 94.3930x; 1.0695x over previous
import functools

import jax
import jax.numpy as jnp
from jax import lax
from jax.experimental import pallas as pl
from jax.experimental.pallas import tpu as pltpu
from jax.experimental.pallas import tpu_sc as plsc

N_NODES = 10000
NPAD = 10240
E_REAL = 320000
EPT = E_REAL // 32
EGROUPS = EPT // 16
ECHUNK = 2000
DOFF = E_REAL
DEGROWS = 79
DUMP = NPAD
SLPT = NPAD // 32
SLG = SLPT // 16
ROWS = 82
GSTRIDE = 128
ATFLAT = NPAD * GSTRIDE
AT_PER_TILE = ATFLAT // 16
ATOUT = N_NODES * GSTRIDE
ATOUT_PER_TILE = ATOUT // 16
DEG_PER_TILE = NPAD // 16
NB = 5
BLK = N_NODES // NB

_mesh = plsc.VectorSubcoreMesh(core_axis_name="c", subcore_axis_name="s")


def _fast_rsqrt(d):
    xi = lax.bitcast_convert_type(d, jnp.int32)
    yi = jnp.int32(0x5F3759DF) - (xi >> 1)
    y = lax.bitcast_convert_type(yi, jnp.float32)
    for _ in range(3):
        y = y * (jnp.float32(1.5) - jnp.float32(0.5) * d * y * y)
    return y


@functools.partial(
    pl.kernel,
    out_type=jax.ShapeDtypeStruct((2 * NPAD,), jnp.float32),
    mesh=_mesh,
    compiler_params=pltpu.CompilerParams(needs_layout_passes=False),
    scratch_types=[
        pltpu.VMEM((ECHUNK,), jnp.int32),
        pltpu.VMEM((DEGROWS, 128), jnp.int32),
        pltpu.VMEM((128,), jnp.float32),
        pltpu.VMEM((DEG_PER_TILE + 8,), jnp.float32),
        pltpu.VMEM_SHARED((NPAD + 128,), jnp.float32),
        pltpu.SemaphoreType.DMA,
    ],
)
def _deg_kernel(bc_hbm, out_hbm, dst_v, idx_v, ones_v, zeros_v, deg_sh, sem):
    c = lax.axis_index("c")
    s = lax.axis_index("s")
    w = c * 16 + s
    base = DOFF + w * EPT

    def zinit_body(i, _):
        zeros_v[pl.ds(i * 16, 16)] = jnp.zeros((16,), jnp.float32)
        return 0

    lax.fori_loop(0, (DEG_PER_TILE + 8) // 16, zinit_body, 0)
    pltpu.sync_copy(zeros_v, deg_sh.at[pl.ds(s * (DEG_PER_TILE + 8),
                                             DEG_PER_TILE + 8)])

    def ones_body(l, _):
        ones_v[pl.ds(l * 16, 16)] = jnp.ones((16,), jnp.float32)
        return 0

    lax.fori_loop(0, 8, ones_body, 0)

    for g0 in range(625, 632):
        idx_v[g0 >> 3, pl.ds((g0 & 7) * 16, 16)] = jnp.full((16,), DUMP,
                                                            jnp.int32)

    plsc.subcore_barrier()

    def fire_body(r, _):
        pltpu.async_copy(ones_v, deg_sh.at[idx_v.at[r]], sem, add=True)
        return 0

    fr = (0, 15, 31, 46, 62, 78)
    for q in range(5):
        pltpu.sync_copy(bc_hbm.at[pl.ds(base + q * ECHUNK, ECHUNK)], dst_v)

        def grp_body(t2, _):
            for u in range(5):
                t = t2 * 5 + u
                gidx = q * 125 + t
                dd = dst_v[pl.ds(t * 16, 16)]
                idx_v[gidx >> 3, pl.ds((gidx & 7) * 16, 16)] = dd
            return 0

        lax.fori_loop(0, 25, grp_body, 0)
        lax.fori_loop(fr[q], fr[q + 1], fire_body, 0)

    lax.fori_loop(78, DEGROWS, fire_body, 0)

    def drain_body(r, _):
        pltpu.make_async_copy(ones_v, deg_sh.at[idx_v.at[r]], sem).wait()
        return 0

    lax.fori_loop(0, DEGROWS, drain_body, 0)
    plsc.subcore_barrier()
    pltpu.sync_copy(
        deg_sh.at[pl.ds(s * DEG_PER_TILE, DEG_PER_TILE)],
        out_hbm.at[pl.ds(c * NPAD + s * DEG_PER_TILE, DEG_PER_TILE)],
    )


@functools.partial(
    pl.kernel,
    out_type=jax.ShapeDtypeStruct((2 * ATOUT,), jnp.float32),
    mesh=_mesh,
    compiler_params=pltpu.CompilerParams(needs_layout_passes=False),
    scratch_types=[
        pltpu.VMEM((ECHUNK,), jnp.int32),
        pltpu.VMEM((ECHUNK,), jnp.int32),
        pltpu.VMEM((ROWS, 128), jnp.int32),
        pltpu.VMEM((ROWS, 128), jnp.float32),
        pltpu.VMEM((NPAD,), jnp.float32),
        pltpu.VMEM((DEG_PER_TILE,), jnp.float32),
        pltpu.VMEM((AT_PER_TILE // 64,), jnp.float32),
        pltpu.VMEM_SHARED((ATFLAT,), jnp.float32),
        pltpu.VMEM_SHARED((NPAD,), jnp.float32),
        pltpu.SemaphoreType.DMA,
    ],
)
def _a_kernel(bc_hbm, deg_hbm, a_out, src_v, dst_v, idx_v, val_v,
              dinv_v, dslc_v, zeros_v, a_sh, dinv_sh, sem):
    c = lax.axis_index("c")
    s = lax.axis_index("s")
    w = c * 16 + s
    base = w * EPT
    zchunk = AT_PER_TILE // 64
    lane1 = jnp.arange(16, dtype=jnp.int32)

    def zinit_body(i, _):
        zeros_v[pl.ds(i * 16, 16)] = jnp.zeros((16,), jnp.float32)
        return 0

    lax.fori_loop(0, zchunk // 16, zinit_body, 0)

    def zcopy_body(t, _):
        pltpu.async_copy(zeros_v, a_sh.at[pl.ds((s * 64 + t) * zchunk,
                                                zchunk)], sem)
        return 0

    lax.fori_loop(0, 64, zcopy_body, 0)

    pltpu.sync_copy(deg_hbm.at[pl.ds(s * DEG_PER_TILE, DEG_PER_TILE)],
                    dslc_v)
    pltpu.sync_copy(deg_hbm.at[pl.ds(NPAD + s * DEG_PER_TILE, DEG_PER_TILE)],
                    dinv_v.at[pl.ds(0, DEG_PER_TILE)])

    def dinv_body(i, _):
        d = (dslc_v[pl.ds(i * 16, 16)] + dinv_v[pl.ds(i * 16, 16)]
             + jnp.float32(1.0))
        dslc_v[pl.ds(i * 16, 16)] = _fast_rsqrt(d)
        return 0

    lax.fori_loop(0, DEG_PER_TILE // 16, dinv_body, 0)
    pltpu.sync_copy(dslc_v, dinv_sh.at[pl.ds(s * DEG_PER_TILE,
                                             DEG_PER_TILE)])

    def zdrain_body(t, _):
        pltpu.make_async_copy(
            zeros_v, a_sh.at[pl.ds((s * 64 + t) * zchunk, zchunk)], sem
        ).wait()
        return 0

    lax.fori_loop(0, 64, zdrain_body, 0)
    plsc.subcore_barrier()
    pltpu.sync_copy(dinv_sh, dinv_v)

    def fire_body(r, _):
        pltpu.async_copy(val_v.at[r], a_sh.at[idx_v.at[r]], sem, add=True)
        return 0

    fr = (0, 15, 31, 46, 62, 78)
    for q in range(5):
        pltpu.sync_copy(bc_hbm.at[pl.ds(base + q * ECHUNK, ECHUNK)], src_v)
        pltpu.sync_copy(bc_hbm.at[pl.ds(DOFF + base + q * ECHUNK, ECHUNK)],
                        dst_v)

        def grp_body(t2, _):
            for u in range(5):
                t = t2 * 5 + u
                gidx = q * 125 + t
                dd = dst_v[pl.ds(t * 16, 16)]
                ss = src_v[pl.ds(t * 16, 16)]
                g = (dd * 5243) >> 19
                idx_v[gidx >> 3, pl.ds((gidx & 7) * 16, 16)] = (ss << 7) + g
                val_v[gidx >> 3, pl.ds((gidx & 7) * 16, 16)] = (
                    plsc.load_gather(dinv_v, [dd])
                    * plsc.load_gather(dinv_v, [ss])
                )
            return 0

        lax.fori_loop(0, 25, grp_body, 0)
        lax.fori_loop(fr[q], fr[q + 1], fire_body, 0)

    for g0 in range(625, 632):
        idx_v[g0 >> 3, pl.ds((g0 & 7) * 16, 16)] = jnp.zeros((16,),
                                                             jnp.int32)
        val_v[g0 >> 3, pl.ds((g0 & 7) * 16, 16)] = jnp.zeros((16,),
                                                             jnp.float32)

    for t in range(SLG):
        gidx = 632 + t
        n = w * SLPT + t * 16 + lane1
        dv = plsc.load_gather(dinv_v, [n])
        g = (n * 5243) >> 19
        idx_v[gidx >> 3, pl.ds((gidx & 7) * 16, 16)] = (n << 7) + g
        val_v[gidx >> 3, pl.ds((gidx & 7) * 16, 16)] = jnp.where(
            n < N_NODES, dv * dv, jnp.float32(0.0))

    for g0 in range(652, 656):
        idx_v[g0 >> 3, pl.ds((g0 & 7) * 16, 16)] = jnp.zeros((16,),
                                                             jnp.int32)
        val_v[g0 >> 3, pl.ds((g0 & 7) * 16, 16)] = jnp.zeros((16,),
                                                             jnp.float32)

    lax.fori_loop(78, ROWS, fire_body, 0)

    def drain_body(r, _):
        pltpu.make_async_copy(val_v.at[r], a_sh.at[idx_v.at[r]], sem).wait()
        return 0

    lax.fori_loop(0, ROWS, drain_body, 0)
    plsc.subcore_barrier()
    pltpu.sync_copy(
        a_sh.at[pl.ds(s * ATOUT_PER_TILE, ATOUT_PER_TILE)],
        a_out.at[pl.ds(c * ATOUT + s * ATOUT_PER_TILE, ATOUT_PER_TILE)],
    )


def _tc_body(x_ref, at_ref, wg_ref, w1_ref, b1_ref, bg_ref, w2_ref, b2_ref,
             out_ref, pool, ssum):
    k = pl.program_id(0)

    @pl.when(k == 0)
    def _():
        pool[...] = jnp.zeros_like(pool)
        ssum[...] = jnp.zeros_like(ssum)

    x = x_ref[...]
    h = jnp.dot(x, wg_ref[...], preferred_element_type=jnp.float32)
    at = at_ref[...]
    asum = at[0] + at[1]
    pool[...] += lax.dot_general(
        asum, h, (((0,), (0,)), ((), ())),
        preferred_element_type=jnp.float32)

    col = lax.broadcasted_iota(jnp.int32, (128, BLK), 1) + k * BLK
    grp = (col * 5243) >> 19
    row = lax.broadcasted_iota(jnp.int32, (128, BLK), 0)
    sel = jnp.where(grp == row, jnp.float32(1.0), jnp.float32(0.0))
    ssum[...] += jnp.dot(sel, x, preferred_element_type=jnp.float32)

    @pl.when(k == NB - 1)
    def _():
        w1a = w1_ref[0:128, :]
        w1b = w1_ref[128:256, :]
        bgw = jnp.dot(bg_ref[...], w1b, preferred_element_type=jnp.float32)
        vpre = (
            jnp.dot(ssum[...], w1a, preferred_element_type=jnp.float32)
            + jnp.dot(pool[...], w1b, preferred_element_type=jnp.float32)
            + b1_ref[...]
            + jnp.float32(100.0) * bgw
        )
        v = jnp.maximum(vpre, jnp.float32(0.0))
        out_ref[...] = (
            jnp.dot(v, w2_ref[...], preferred_element_type=jnp.float32)
            + b2_ref[...]
        )


def kernel(state, edge_index, Wg, bg, W1, b1, W2, b2):
    bc = edge_index.astype(jnp.int32).reshape(-1)

    deg = _deg_kernel(bc)
    at_parts = _a_kernel(bc, deg)
    at3 = at_parts.reshape(2, N_NODES, GSTRIDE)

    out = pl.pallas_call(
        _tc_body,
        grid=(NB,),
        in_specs=[
            pl.BlockSpec((BLK, 128), lambda k: (k, 0)),
            pl.BlockSpec((2, BLK, 128), lambda k: (0, k, 0)),
            pl.BlockSpec((128, 128), lambda k: (0, 0)),
            pl.BlockSpec((256, 128), lambda k: (0, 0)),
            pl.BlockSpec((1, 128), lambda k: (0, 0)),
            pl.BlockSpec((1, 128), lambda k: (0, 0)),
            pl.BlockSpec((128, 1), lambda k: (0, 0)),
            pl.BlockSpec((1, 1), lambda k: (0, 0)),
        ],
        out_specs=pl.BlockSpec((128, 1), lambda k: (0, 0)),
        out_shape=jax.ShapeDtypeStruct((128, 1), jnp.float32),
        scratch_shapes=[
            pltpu.VMEM((128, 128), jnp.float32),
            pltpu.VMEM((128, 128), jnp.float32),
        ],
    )(state, at3, Wg, W1, b1.reshape(1, 128), bg.reshape(1, 128), W2,
      b2.reshape(1, 1))
    return out[:100, 0]


# speedup vs baseline: 95.0767x; 1.0072x over previous
import functools

import jax
import jax.numpy as jnp
from jax import lax
from jax.experimental import pallas as pl
from jax.experimental.pallas import tpu as pltpu
from jax.experimental.pallas import tpu_sc as plsc

N_NODES = 10000
NPAD = 10240
E_REAL = 320000
EPT = E_REAL // 32
EGROUPS = EPT // 16
ECHUNK = 2000
DOFF = E_REAL
DEGROWS = 79
DUMP = NPAD
SLPT = NPAD // 32
SLG = SLPT // 16
ROWS = 82
GSTRIDE = 128
ATFLAT = NPAD * GSTRIDE
AT_PER_TILE = ATFLAT // 16
ATOUT = N_NODES * GSTRIDE
ATOUT_PER_TILE = ATOUT // 16
DEG_PER_TILE = NPAD // 16
NB = 5
BLK = N_NODES // NB

_mesh = plsc.VectorSubcoreMesh(core_axis_name="c", subcore_axis_name="s")


def _fast_rsqrt(d):
    xi = lax.bitcast_convert_type(d, jnp.int32)
    yi = jnp.int32(0x5F3759DF) - (xi >> 1)
    y = lax.bitcast_convert_type(yi, jnp.float32)
    for _ in range(3):
        y = y * (jnp.float32(1.5) - jnp.float32(0.5) * d * y * y)
    return y


@functools.partial(
    pl.kernel,
    out_type=jax.ShapeDtypeStruct((2 * NPAD,), jnp.float32),
    mesh=_mesh,
    compiler_params=pltpu.CompilerParams(needs_layout_passes=False),
    scratch_types=[
        pltpu.VMEM((ECHUNK,), jnp.int32),
        pltpu.VMEM((DEGROWS, 128), jnp.int32),
        pltpu.VMEM((128,), jnp.float32),
        pltpu.VMEM((DEG_PER_TILE + 16,), jnp.float32),
        pltpu.VMEM_SHARED((NPAD + 128,), jnp.float32),
        pltpu.SemaphoreType.DMA,
    ],
)
def _deg_kernel(bc_hbm, out_hbm, dst_v, idx_v, ones_v, zeros_v, deg_sh, sem):
    c = lax.axis_index("c")
    s = lax.axis_index("s")
    w = c * 16 + s
    base = DOFF + w * EPT

    def zinit_body(i, _):
        zeros_v[pl.ds(i * 16, 16)] = jnp.zeros((16,), jnp.float32)
        return 0

    lax.fori_loop(0, (DEG_PER_TILE + 16) // 16, zinit_body, 0)
    pltpu.sync_copy(zeros_v.at[pl.ds(0, DEG_PER_TILE + 8)],
                    deg_sh.at[pl.ds(s * (DEG_PER_TILE + 8),
                                    DEG_PER_TILE + 8)])

    def ones_body(l, _):
        ones_v[pl.ds(l * 16, 16)] = jnp.ones((16,), jnp.float32)
        return 0

    lax.fori_loop(0, 8, ones_body, 0)

    for g0 in range(625, 632):
        idx_v[g0 >> 3, pl.ds((g0 & 7) * 16, 16)] = jnp.full((16,), DUMP,
                                                            jnp.int32)

    plsc.subcore_barrier()

    def fire_body(r, _):
        pltpu.async_copy(ones_v, deg_sh.at[idx_v.at[r]], sem, add=True)
        return 0

    fr = (0, 15, 31, 46, 62, 78)
    for q in range(5):
        pltpu.sync_copy(bc_hbm.at[pl.ds(base + q * ECHUNK, ECHUNK)], dst_v)

        def grp_body(t2, _):
            for u in range(5):
                t = t2 * 5 + u
                gidx = q * 125 + t
                dd = dst_v[pl.ds(t * 16, 16)]
                idx_v[gidx >> 3, pl.ds((gidx & 7) * 16, 16)] = dd
            return 0

        lax.fori_loop(0, 25, grp_body, 0)
        lax.fori_loop(fr[q], fr[q + 1], fire_body, 0)

    lax.fori_loop(78, DEGROWS, fire_body, 0)

    def drain_body(r, _):
        pltpu.make_async_copy(ones_v, deg_sh.at[idx_v.at[r]], sem).wait()
        return 0

    lax.fori_loop(0, DEGROWS, drain_body, 0)
    plsc.subcore_barrier()
    pltpu.sync_copy(
        deg_sh.at[pl.ds(s * DEG_PER_TILE, DEG_PER_TILE)],
        out_hbm.at[pl.ds(c * NPAD + s * DEG_PER_TILE, DEG_PER_TILE)],
    )


@functools.partial(
    pl.kernel,
    out_type=jax.ShapeDtypeStruct((2 * ATOUT,), jnp.float32),
    mesh=_mesh,
    compiler_params=pltpu.CompilerParams(needs_layout_passes=False),
    scratch_types=[
        pltpu.VMEM((ECHUNK,), jnp.int32),
        pltpu.VMEM((ECHUNK,), jnp.int32),
        pltpu.VMEM((ROWS, 128), jnp.int32),
        pltpu.VMEM((ROWS, 128), jnp.float32),
        pltpu.VMEM((NPAD,), jnp.float32),
        pltpu.VMEM((DEG_PER_TILE,), jnp.float32),
        pltpu.VMEM((AT_PER_TILE // 64,), jnp.float32),
        pltpu.VMEM_SHARED((ATFLAT,), jnp.float32),
        pltpu.VMEM_SHARED((NPAD,), jnp.float32),
        pltpu.SemaphoreType.DMA,
    ],
)
def _a_kernel(bc_hbm, deg_hbm, a_out, src_v, dst_v, idx_v, val_v,
              dinv_v, dslc_v, zeros_v, a_sh, dinv_sh, sem):
    c = lax.axis_index("c")
    s = lax.axis_index("s")
    w = c * 16 + s
    base = w * EPT
    zchunk = AT_PER_TILE // 64
    lane1 = jnp.arange(16, dtype=jnp.int32)

    def zinit_body(i, _):
        zeros_v[pl.ds(i * 16, 16)] = jnp.zeros((16,), jnp.float32)
        return 0

    lax.fori_loop(0, zchunk // 16, zinit_body, 0)

    def zcopy_body(t, _):
        pltpu.async_copy(zeros_v, a_sh.at[pl.ds((s * 64 + t) * zchunk,
                                                zchunk)], sem)
        return 0

    lax.fori_loop(0, 64, zcopy_body, 0)

    pltpu.sync_copy(deg_hbm.at[pl.ds(s * DEG_PER_TILE, DEG_PER_TILE)],
                    dslc_v)
    pltpu.sync_copy(deg_hbm.at[pl.ds(NPAD + s * DEG_PER_TILE, DEG_PER_TILE)],
                    dinv_v.at[pl.ds(0, DEG_PER_TILE)])

    def dinv_body(i, _):
        d = (dslc_v[pl.ds(i * 16, 16)] + dinv_v[pl.ds(i * 16, 16)]
             + jnp.float32(1.0))
        dslc_v[pl.ds(i * 16, 16)] = _fast_rsqrt(d)
        return 0

    lax.fori_loop(0, DEG_PER_TILE // 16, dinv_body, 0)
    pltpu.sync_copy(dslc_v, dinv_sh.at[pl.ds(s * DEG_PER_TILE,
                                             DEG_PER_TILE)])

    def zdrain_body(t, _):
        pltpu.make_async_copy(
            zeros_v, a_sh.at[pl.ds((s * 64 + t) * zchunk, zchunk)], sem
        ).wait()
        return 0

    lax.fori_loop(0, 64, zdrain_body, 0)
    plsc.subcore_barrier()
    pltpu.sync_copy(dinv_sh, dinv_v)

    def fire_body(r, _):
        pltpu.async_copy(val_v.at[r], a_sh.at[idx_v.at[r]], sem, add=True)
        return 0

    fr = (0, 15, 31, 46, 62, 78)
    for q in range(5):
        pltpu.sync_copy(bc_hbm.at[pl.ds(base + q * ECHUNK, ECHUNK)], src_v)
        pltpu.sync_copy(bc_hbm.at[pl.ds(DOFF + base + q * ECHUNK, ECHUNK)],
                        dst_v)

        def grp_body(t2, _):
            for u in range(5):
                t = t2 * 5 + u
                gidx = q * 125 + t
                dd = dst_v[pl.ds(t * 16, 16)]
                ss = src_v[pl.ds(t * 16, 16)]
                g = (dd * 5243) >> 19
                idx_v[gidx >> 3, pl.ds((gidx & 7) * 16, 16)] = (ss << 7) + g
                val_v[gidx >> 3, pl.ds((gidx & 7) * 16, 16)] = (
                    plsc.load_gather(dinv_v, [dd])
                    * plsc.load_gather(dinv_v, [ss])
                )
            return 0

        lax.fori_loop(0, 25, grp_body, 0)
        lax.fori_loop(fr[q], fr[q + 1], fire_body, 0)

    for g0 in range(625, 632):
        idx_v[g0 >> 3, pl.ds((g0 & 7) * 16, 16)] = jnp.zeros((16,),
                                                             jnp.int32)
        val_v[g0 >> 3, pl.ds((g0 & 7) * 16, 16)] = jnp.zeros((16,),
                                                             jnp.float32)

    for t in range(SLG):
        gidx = 632 + t
        n = w * SLPT + t * 16 + lane1
        dv = plsc.load_gather(dinv_v, [n])
        g = (n * 5243) >> 19
        idx_v[gidx >> 3, pl.ds((gidx & 7) * 16, 16)] = (n << 7) + g
        val_v[gidx >> 3, pl.ds((gidx & 7) * 16, 16)] = jnp.where(
            n < N_NODES, dv * dv, jnp.float32(0.0))

    for g0 in range(652, 656):
        idx_v[g0 >> 3, pl.ds((g0 & 7) * 16, 16)] = jnp.zeros((16,),
                                                             jnp.int32)
        val_v[g0 >> 3, pl.ds((g0 & 7) * 16, 16)] = jnp.zeros((16,),
                                                             jnp.float32)

    lax.fori_loop(78, ROWS, fire_body, 0)

    def drain_body(r, _):
        pltpu.make_async_copy(val_v.at[r], a_sh.at[idx_v.at[r]], sem).wait()
        return 0

    lax.fori_loop(0, ROWS, drain_body, 0)
    plsc.subcore_barrier()
    pltpu.sync_copy(
        a_sh.at[pl.ds(s * ATOUT_PER_TILE, ATOUT_PER_TILE)],
        a_out.at[pl.ds(c * ATOUT + s * ATOUT_PER_TILE, ATOUT_PER_TILE)],
    )


def _tc_body(x_ref, at_ref, wg_ref, w1_ref, b1_ref, bg_ref, w2_ref, b2_ref,
             out_ref, pool, ssum):
    k = pl.program_id(0)

    @pl.when(k == 0)
    def _():
        pool[...] = jnp.zeros_like(pool)
        ssum[...] = jnp.zeros_like(ssum)

    x = x_ref[...]
    h = jnp.dot(x, wg_ref[...], preferred_element_type=jnp.float32)
    at = at_ref[...]
    asum = at[0] + at[1]
    pool[...] += lax.dot_general(
        asum, h, (((0,), (0,)), ((), ())),
        preferred_element_type=jnp.float32)

    col = lax.broadcasted_iota(jnp.int32, (128, BLK), 1) + k * BLK
    grp = (col * 5243) >> 19
    row = lax.broadcasted_iota(jnp.int32, (128, BLK), 0)
    sel = jnp.where(grp == row, jnp.float32(1.0), jnp.float32(0.0))
    ssum[...] += jnp.dot(sel, x, preferred_element_type=jnp.float32)

    @pl.when(k == NB - 1)
    def _():
        w1a = w1_ref[0:128, :]
        w1b = w1_ref[128:256, :]
        bgw = jnp.dot(bg_ref[...], w1b, preferred_element_type=jnp.float32)
        vpre = (
            jnp.dot(ssum[...], w1a, preferred_element_type=jnp.float32)
            + jnp.dot(pool[...], w1b, preferred_element_type=jnp.float32)
            + b1_ref[...]
            + jnp.float32(100.0) * bgw
        )
        v = jnp.maximum(vpre, jnp.float32(0.0))
        out_ref[...] = (
            jnp.dot(v, w2_ref[...], preferred_element_type=jnp.float32)
            + b2_ref[...]
        )


def kernel(state, edge_index, Wg, bg, W1, b1, W2, b2):
    bc = edge_index.astype(jnp.int32).reshape(-1)

    deg = _deg_kernel(bc)
    at_parts = _a_kernel(bc, deg)
    at3 = at_parts.reshape(2, N_NODES, GSTRIDE)

    out = pl.pallas_call(
        _tc_body,
        grid=(NB,),
        in_specs=[
            pl.BlockSpec((BLK, 128), lambda k: (k, 0)),
            pl.BlockSpec((2, BLK, 128), lambda k: (0, k, 0)),
            pl.BlockSpec((128, 128), lambda k: (0, 0)),
            pl.BlockSpec((256, 128), lambda k: (0, 0)),
            pl.BlockSpec((1, 128), lambda k: (0, 0)),
            pl.BlockSpec((1, 128), lambda k: (0, 0)),
            pl.BlockSpec((128, 1), lambda k: (0, 0)),
            pl.BlockSpec((1, 1), lambda k: (0, 0)),
        ],
        out_specs=pl.BlockSpec((128, 1), lambda k: (0, 0)),
        out_shape=jax.ShapeDtypeStruct((128, 1), jnp.float32),
        scratch_shapes=[
            pltpu.VMEM((128, 128), jnp.float32),
            pltpu.VMEM((128, 128), jnp.float32),
        ],
    )(state, at3, Wg, W1, b1.reshape(1, 128), bg.reshape(1, 128), W2,
      b2.reshape(1, 1))
    return out[:100, 0]
